# Initial kernel scaffold; baseline (speedup 1.0000x reference)
import jax
import jax.numpy as jnp
from jax.experimental import pallas as pl


def kernel(Adjacency, node_features, W, b):
    raise NotImplementedError("write your pallas kernel here")


# speedup vs baseline: 9.0612x; 9.0612x over previous
from functools import partial

import jax
import jax.numpy as jnp
from jax.experimental import pallas as pl


def _edgeconv_body(A_ref, Z_ref, R_ref, W_ref, bT_ref, o_ref, *, tb):
    step = pl.program_id(0)
    n = Z_ref.shape[0]
    W = W_ref[...]
    bT = bT_ref[...]
    Z = Z_ref[...]
    iota = jax.lax.broadcasted_iota(jnp.int32, (1, n), 1)
    neg = jnp.float32(-jnp.inf)
    hi = jax.lax.Precision.HIGHEST

    for t in range(tb):
        cc = Z + R_ref[t:t + 1, :]
        msgT = jax.lax.dot_general(W, cc, (((0,), (1,)), ((), ()))) + bT
        norm = jnp.sqrt(jnp.sum(msgT * msgT, axis=0, keepdims=True))
        arow = A_ref[t:t + 1, :]
        m = (arow != 0) & (iota != step * tb + t)
        sc = jnp.where(m, norm, neg)
        mx = jnp.max(sc)
        cand = jnp.where(m & (norm == mx), iota, jnp.int32(n))
        jstar = jnp.min(cand)
        onehot = (iota == jstar).astype(jnp.float32)
        sel = jax.lax.dot_general(onehot, msgT, (((1,), (1,)), ((), ())),
                                  precision=hi)
        o_ref[t:t + 1, :] = sel


def kernel(Adjacency, node_features, W, b):
    n, d = node_features.shape
    dout = W.shape[-1]
    tb = 8
    x = node_features.astype(jnp.float32)
    Z = jnp.concatenate([jnp.zeros_like(x), x], axis=1)
    R = jnp.concatenate([x, -x], axis=1)
    bT = b.reshape(dout, 1).astype(jnp.float32)
    return pl.pallas_call(
        partial(_edgeconv_body, tb=tb),
        grid=(n // tb,),
        in_specs=[
            pl.BlockSpec((tb, n), lambda i: (i, 0)),
            pl.BlockSpec((n, 2 * d), lambda i: (0, 0)),
            pl.BlockSpec((tb, 2 * d), lambda i: (i, 0)),
            pl.BlockSpec((2 * d, dout), lambda i: (0, 0)),
            pl.BlockSpec((dout, 1), lambda i: (0, 0)),
        ],
        out_specs=pl.BlockSpec((tb, dout), lambda i: (i, 0)),
        out_shape=jax.ShapeDtypeStruct((n, dout), jnp.float32),
    )(Adjacency, Z, R, W, bT)
